# baseline (speedup 1.0000x reference)
import numpy as np
import jax
import jax.numpy as jnp
from jax.experimental import pallas as pl
from jax.experimental.pallas import tpu as pltpu

B = 8
IN_FEATS = 256
HID = 512
NUM_CLASSES = 1000
NUM_NODES = 16
NODE_DIM = IN_FEATS // NUM_NODES
H = W = 224
KPAD = 32


def _conv_mean_body(x_ref, w_ref, o_ref):
    def block_step(i, acc):
        y0 = pl.multiple_of(i * 8, 8)
        win = x_ref[0, pl.ds(y0, 10), :, :]
        for r in range(8):
            pt = win[r:r + 3].reshape(48, 256)
            z = jax.lax.dot_general(
                pt, w_ref[...],
                dimension_numbers=(((0,), (0,)), ((), ())),
                preferred_element_type=jnp.float32)
            acc = acc + jnp.sum(
                jnp.maximum(z, 0.0).reshape(32, 8, IN_FEATS), axis=0)
        return acc

    acc = jax.lax.fori_loop(0, H // 8, block_step,
                            jnp.zeros((8, IN_FEATS), jnp.float32))
    o_ref[0, 0, :] = jnp.sum(acc, axis=0) * jnp.float32(1.0 / (H * W))


def _gcn_tail_body(nodes_ref, a_ref, p_ref, w1t_ref, b1_ref, w2t_ref, b2_ref,
                   wfct_ref, bfc_ref, o_ref):
    f32 = jnp.float32
    nodes = nodes_ref[...]
    agg1 = jax.lax.dot_general(
        a_ref[...], nodes, (((1,), (0,)), ((), ())), preferred_element_type=f32)
    h1 = jnp.maximum(
        jax.lax.dot_general(agg1, w1t_ref[...], (((1,), (0,)), ((), ())),
                            preferred_element_type=f32) + b1_ref[...], 0.0)
    agg2 = jax.lax.dot_general(
        a_ref[...], h1, (((1,), (0,)), ((), ())), preferred_element_type=f32)
    h2 = jnp.maximum(
        jax.lax.dot_general(agg2, w2t_ref[...], (((1,), (0,)), ((), ())),
                            preferred_element_type=f32) + b2_ref[...], 0.0)
    pooled = jax.lax.dot_general(
        p_ref[...], h2, (((1,), (0,)), ((), ())), preferred_element_type=f32)
    o_ref[...] = jax.lax.dot_general(
        pooled, wfct_ref[...], (((1,), (0,)), ((), ())),
        preferred_element_type=f32) + bfc_ref[...]


def _agg_matrix():
    a16 = np.zeros((NUM_NODES, NUM_NODES), np.float32)
    for i in range(NUM_NODES - 1):
        a16[i, i + 1:] = 1.0 / (NUM_NODES - 1 - i)
    a = np.kron(np.eye(B, dtype=np.float32), a16)
    return jnp.asarray(a)


def _pool_matrix():
    p = np.kron(np.eye(B, dtype=np.float32),
                np.full((1, NUM_NODES), 1.0 / NUM_NODES, np.float32))
    return jnp.asarray(p)


def kernel(x, conv_w, conv_b, w1, b1, w2, b2, wfc, bfc):
    xp = jnp.pad(x, ((0, 0), (0, 0), (1, 15), (1, 1)))
    planes = [xp[:, ci, :, dx:dx + W] for ci in range(3) for dx in range(3)]
    planes.append(jnp.ones_like(planes[0]))
    xst = jnp.stack(planes, axis=2)
    xst = jnp.pad(xst, ((0, 0), (0, 0), (0, 6), (0, 32))).astype(jnp.bfloat16)

    w3 = conv_w.transpose(2, 1, 3, 0).reshape(3, 9, IN_FEATS)
    w3 = jnp.pad(w3, ((0, 0), (0, 7), (0, 0)))
    w3 = w3.at[0, 9, :].set(conv_b)
    wmat = w3.reshape(48, IN_FEATS).astype(jnp.bfloat16)

    h = pl.pallas_call(
        _conv_mean_body,
        grid=(B,),
        in_specs=[
            pl.BlockSpec((1, 240, 16, 256), lambda i: (i, 0, 0, 0)),
            pl.BlockSpec((48, IN_FEATS), lambda i: (0, 0)),
        ],
        out_specs=pl.BlockSpec((1, 1, IN_FEATS), lambda i: (i, 0, 0)),
        out_shape=jax.ShapeDtypeStruct((B, 1, IN_FEATS), jnp.float32),
        compiler_params=pltpu.CompilerParams(
            dimension_semantics=("parallel",)),
    )(xst, wmat)

    nodes = h.reshape(B * NUM_NODES, NODE_DIM)

    out = pl.pallas_call(
        _gcn_tail_body,
        out_shape=jax.ShapeDtypeStruct((B, NUM_CLASSES), jnp.float32),
    )(nodes, _agg_matrix(), _pool_matrix(),
      w1.T, b1[None, :], w2.T, b2[None, :], wfc.T, bfc[None, :])
    return out


# speedup vs baseline: 1.0559x; 1.0559x over previous
import numpy as np
import jax
import jax.numpy as jnp
from jax.experimental import pallas as pl
from jax.experimental.pallas import tpu as pltpu

B = 8
IN_FEATS = 256
HID = 512
NUM_CLASSES = 1000
NUM_NODES = 16
NODE_DIM = IN_FEATS // NUM_NODES
H = W = 224
KPAD = 32


def _conv_mean_body(x_ref, w_ref, o_ref, acc_ref):
    acc_ref[...] = jnp.zeros((IN_FEATS, 128), jnp.float32)

    def block_step(i, carry):
        y0 = pl.multiple_of(i * 8, 8)
        win = x_ref[0, pl.ds(y0, 10), :, :]
        for r in range(8):
            pt = win[r:r + 3].reshape(48, 256)
            zt = jax.lax.dot_general(
                w_ref[...], pt,
                dimension_numbers=(((1,), (0,)), ((), ())),
                preferred_element_type=jnp.float32)
            zr = jnp.maximum(zt, 0.0)
            acc_ref[...] += zr[:, :128] + zr[:, 128:]
        return carry

    jax.lax.fori_loop(0, H // 8, block_step, 0)
    o_ref[0, 0, :] = jnp.sum(acc_ref[...], axis=1) * jnp.float32(1.0 / (H * W))


def _gcn_tail_body(nodes_ref, a_ref, p_ref, w1t_ref, b1_ref, w2t_ref, b2_ref,
                   wfct_ref, bfc_ref, o_ref):
    f32 = jnp.float32
    nodes = nodes_ref[...]
    agg1 = jax.lax.dot_general(
        a_ref[...], nodes, (((1,), (0,)), ((), ())), preferred_element_type=f32)
    h1 = jnp.maximum(
        jax.lax.dot_general(agg1, w1t_ref[...], (((1,), (0,)), ((), ())),
                            preferred_element_type=f32) + b1_ref[...], 0.0)
    agg2 = jax.lax.dot_general(
        a_ref[...], h1, (((1,), (0,)), ((), ())), preferred_element_type=f32)
    h2 = jnp.maximum(
        jax.lax.dot_general(agg2, w2t_ref[...], (((1,), (0,)), ((), ())),
                            preferred_element_type=f32) + b2_ref[...], 0.0)
    pooled = jax.lax.dot_general(
        p_ref[...], h2, (((1,), (0,)), ((), ())), preferred_element_type=f32)
    o_ref[...] = jax.lax.dot_general(
        pooled, wfct_ref[...], (((1,), (0,)), ((), ())),
        preferred_element_type=f32) + bfc_ref[...]


def _agg_matrix():
    a16 = np.zeros((NUM_NODES, NUM_NODES), np.float32)
    for i in range(NUM_NODES - 1):
        a16[i, i + 1:] = 1.0 / (NUM_NODES - 1 - i)
    a = np.kron(np.eye(B, dtype=np.float32), a16)
    return jnp.asarray(a)


def _pool_matrix():
    p = np.kron(np.eye(B, dtype=np.float32),
                np.full((1, NUM_NODES), 1.0 / NUM_NODES, np.float32))
    return jnp.asarray(p)


def kernel(x, conv_w, conv_b, w1, b1, w2, b2, wfc, bfc):
    xp = jnp.pad(x, ((0, 0), (0, 0), (1, 15), (1, 1)))
    planes = [xp[:, ci, :, dx:dx + W] for ci in range(3) for dx in range(3)]
    planes.append(jnp.ones_like(planes[0]))
    xst = jnp.stack(planes, axis=2)
    xst = jnp.pad(xst, ((0, 0), (0, 0), (0, 6), (0, 32))).astype(jnp.bfloat16)

    w3 = conv_w.transpose(2, 1, 3, 0).reshape(3, 9, IN_FEATS)
    w3 = jnp.pad(w3, ((0, 0), (0, 7), (0, 0)))
    w3 = w3.at[0, 9, :].set(conv_b)
    wmat = w3.reshape(48, IN_FEATS).T.astype(jnp.bfloat16)

    h = pl.pallas_call(
        _conv_mean_body,
        grid=(B,),
        in_specs=[
            pl.BlockSpec((1, 240, 16, 256), lambda i: (i, 0, 0, 0)),
            pl.BlockSpec((IN_FEATS, 48), lambda i: (0, 0)),
        ],
        scratch_shapes=[pltpu.VMEM((IN_FEATS, 128), jnp.float32)],
        out_specs=pl.BlockSpec((1, 1, IN_FEATS), lambda i: (i, 0, 0)),
        out_shape=jax.ShapeDtypeStruct((B, 1, IN_FEATS), jnp.float32),
        compiler_params=pltpu.CompilerParams(
            dimension_semantics=("parallel",)),
    )(xst, wmat)

    nodes = h.reshape(B * NUM_NODES, NODE_DIM)

    out = pl.pallas_call(
        _gcn_tail_body,
        out_shape=jax.ShapeDtypeStruct((B, NUM_CLASSES), jnp.float32),
    )(nodes, _agg_matrix(), _pool_matrix(),
      w1.T, b1[None, :], w2.T, b2[None, :], wfc.T, bfc[None, :])
    return out


# speedup vs baseline: 1.8424x; 1.7448x over previous
import numpy as np
import jax
import jax.numpy as jnp
from jax.experimental import pallas as pl
from jax.experimental.pallas import tpu as pltpu

B = 8
IN_FEATS = 256
HID = 512
NUM_CLASSES = 1000
NUM_NODES = 16
NODE_DIM = IN_FEATS // NUM_NODES
H = W = 224
KPAD = 32


def _conv_mean_body(x_ref, w_ref, o_ref):
    zrows = jnp.zeros((2, 256), jnp.bfloat16)

    def block_step(i, acc):
        y0 = pl.multiple_of(i * 8, 8)
        win = [x_ref[0, j, pl.ds(y0, 16), :] for j in range(10)]
        for r in range(8):
            pt = jnp.concatenate([wj[r:r + 3] for wj in win] + [zrows],
                                 axis=0)
            z = jax.lax.dot_general(
                pt, w_ref[...],
                dimension_numbers=(((0,), (0,)), ((), ())),
                preferred_element_type=jnp.float32)
            acc = acc + jnp.sum(
                jnp.maximum(z, 0.0).reshape(32, 8, IN_FEATS), axis=0)
        return acc

    acc = jax.lax.fori_loop(0, H // 8, block_step,
                            jnp.zeros((8, IN_FEATS), jnp.float32))
    o_ref[0, 0, :] = jnp.sum(acc, axis=0) * jnp.float32(1.0 / (H * W))


def _gcn_tail_body(nodes_ref, a_ref, p_ref, w1t_ref, b1_ref, w2t_ref, b2_ref,
                   wfct_ref, bfc_ref, o_ref):
    f32 = jnp.float32
    nodes = nodes_ref[...]
    agg1 = jax.lax.dot_general(
        a_ref[...], nodes, (((1,), (0,)), ((), ())), preferred_element_type=f32)
    h1 = jnp.maximum(
        jax.lax.dot_general(agg1, w1t_ref[...], (((1,), (0,)), ((), ())),
                            preferred_element_type=f32) + b1_ref[...], 0.0)
    agg2 = jax.lax.dot_general(
        a_ref[...], h1, (((1,), (0,)), ((), ())), preferred_element_type=f32)
    h2 = jnp.maximum(
        jax.lax.dot_general(agg2, w2t_ref[...], (((1,), (0,)), ((), ())),
                            preferred_element_type=f32) + b2_ref[...], 0.0)
    pooled = jax.lax.dot_general(
        p_ref[...], h2, (((1,), (0,)), ((), ())), preferred_element_type=f32)
    o_ref[...] = jax.lax.dot_general(
        pooled, wfct_ref[...], (((1,), (0,)), ((), ())),
        preferred_element_type=f32) + bfc_ref[...]


def _agg_matrix():
    a16 = np.zeros((NUM_NODES, NUM_NODES), np.float32)
    for i in range(NUM_NODES - 1):
        a16[i, i + 1:] = 1.0 / (NUM_NODES - 1 - i)
    a = np.kron(np.eye(B, dtype=np.float32), a16)
    return jnp.asarray(a)


def _pool_matrix():
    p = np.kron(np.eye(B, dtype=np.float32),
                np.full((1, NUM_NODES), 1.0 / NUM_NODES, np.float32))
    return jnp.asarray(p)


def kernel(x, conv_w, conv_b, w1, b1, w2, b2, wfc, bfc):
    xp = jnp.pad(x, ((0, 0), (0, 0), (1, 15), (1, 1)))
    planes = [xp[:, ci, :, dx:dx + W] for ci in range(3) for dx in range(3)]
    planes.append(jnp.ones_like(planes[0]))
    xs = jnp.stack(planes, axis=1)
    xs = jnp.pad(xs, ((0, 0), (0, 0), (0, 0), (0, 32))).astype(jnp.bfloat16)

    wmat = conv_w.transpose(1, 3, 2, 0).reshape(27, IN_FEATS)
    wmat = jnp.concatenate(
        [wmat, conv_b[None, :], jnp.zeros((4, IN_FEATS), conv_b.dtype)], axis=0)
    wmat = wmat.astype(jnp.bfloat16)

    h = pl.pallas_call(
        _conv_mean_body,
        grid=(B,),
        in_specs=[
            pl.BlockSpec((1, 10, 240, 256), lambda i: (i, 0, 0, 0)),
            pl.BlockSpec((KPAD, IN_FEATS), lambda i: (0, 0)),
        ],
        out_specs=pl.BlockSpec((1, 1, IN_FEATS), lambda i: (i, 0, 0)),
        out_shape=jax.ShapeDtypeStruct((B, 1, IN_FEATS), jnp.float32),
        compiler_params=pltpu.CompilerParams(
            dimension_semantics=("parallel",)),
    )(xs, wmat)

    nodes = h.reshape(B * NUM_NODES, NODE_DIM)

    out = pl.pallas_call(
        _gcn_tail_body,
        out_shape=jax.ShapeDtypeStruct((B, NUM_CLASSES), jnp.float32),
    )(nodes, _agg_matrix(), _pool_matrix(),
      w1.T, b1[None, :], w2.T, b2[None, :], wfc.T, bfc[None, :])
    return out
